# baseline (speedup 1.0000x reference)
import jax
import jax.numpy as jnp
from jax import lax
from jax.experimental import pallas as pl
from jax.experimental.pallas import tpu as pltpu
from jax.experimental.pallas import tpu_sc as plsc

_NC, _NS, _L = 2, 16, 16
_NW = _NC * _NS
_B = 16384
_BPW = _B // _NW
_D = 32
_C = 128
_NCHUNK = _BPW // _C
_GLOBAL_MEAN = 3.5

_TC = 7813
_GRP = 13
_NGRP = _TC // _GRP
_BAND = _TC * 1024
_ROWS = 4 * _TC * 8
_FLAT = _ROWS * 128


def _retile_body(in_ref, out_ref):
    x = in_ref[...]
    out_ref[...] = x.reshape(8, _GRP, 128).swapaxes(0, 1).reshape(_GRP * 8, 128)


def _retile(table):
    tt = jnp.swapaxes(table, 0, 1)
    return pl.pallas_call(
        _retile_body,
        grid=(4, _NGRP),
        in_specs=[pl.BlockSpec((8, _GRP * 128), lambda tr, g: (tr, g))],
        out_specs=pl.BlockSpec((_GRP * 8, 128), lambda tr, g: (tr * _NGRP + g, 0)),
        out_shape=jax.ShapeDtypeStruct((_ROWS, 128), jnp.float32),
    )(tt)


def _body(in_hbm, ut_hbm, it_hbm, ub_hbm, ib_hbm, out_hbm,
          uidx, iidx, ubase, ibase, uidxm, iidxm, ures, ires,
          ubias, ibias, acc, sem):
    wid = lax.axis_index("s") * _NC + lax.axis_index("c")
    base = wid * _BPW

    pltpu.sync_copy(in_hbm.at[0].at[pl.ds(base, _BPW)], uidx)
    pltpu.sync_copy(in_hbm.at[1].at[pl.ds(base, _BPW)], iidx)

    for q in range(_NCHUNK):
        csl = pl.ds(q * _C, _C)

        @pl.loop(0, _C // _L)
        def _bofs(b):
            s16 = pl.ds(q * _C + b * _L, _L)
            d16 = pl.ds(b * _L, _L)
            u = uidx[s16]
            i = iidx[s16]
            ubase[d16] = (jnp.left_shift(jnp.right_shift(u, 7), 10)
                          + jnp.bitwise_and(u, 127))
            ibase[d16] = (jnp.left_shift(jnp.right_shift(i, 7), 10)
                          + jnp.bitwise_and(i, 127))

        @pl.loop(0, _D)
        def _rows(r):
            off = (jnp.right_shift(r, 3) * _BAND
                   + jnp.left_shift(jnp.bitwise_and(r, 7), 7))

            @pl.loop(0, _C // _L)
            def _seg(b):
                d16 = pl.ds(b * _L, _L)
                uidxm[r, d16] = ubase[d16] + off
                iidxm[r, d16] = ibase[d16] + off

        copies = [
            pltpu.async_copy(ub_hbm.at[0].at[uidx.at[csl]], ubias, sem),
            pltpu.async_copy(ib_hbm.at[0].at[iidx.at[csl]], ibias, sem),
        ]
        for r in range(_D):
            copies.append(
                pltpu.async_copy(ut_hbm.at[0].at[uidxm.at[r]], ures.at[r], sem))
            copies.append(
                pltpu.async_copy(it_hbm.at[0].at[iidxm.at[r]], ires.at[r], sem))
        for c in copies:
            c.wait()

        @pl.loop(0, _C // _L)
        def _dot(b):
            d16 = pl.ds(b * _L, _L)
            accv = ubias[d16] + ibias[d16] + _GLOBAL_MEAN
            for f in range(_D):
                accv = accv + ures[f, d16] * ires[f, d16]
            acc[pl.ds(q * _C + b * _L, _L)] = accv

    pltpu.sync_copy(acc, out_hbm.at[pl.ds(base, _BPW)])


def kernel(inputs, user_table, item_table, user_bias_table, item_bias_table):
    inputs_t = inputs.T.astype(jnp.int32)
    ut_f = _retile(user_table).reshape(1, _FLAT)
    it_f = _retile(item_table).reshape(1, _FLAT)
    mesh = plsc.VectorSubcoreMesh(core_axis_name="c", subcore_axis_name="s")
    run = pl.kernel(
        _body,
        out_type=jax.ShapeDtypeStruct((_B,), jnp.float32),
        mesh=mesh,
        scratch_types=[
            pltpu.VMEM((_BPW,), jnp.int32),
            pltpu.VMEM((_BPW,), jnp.int32),
            pltpu.VMEM((_C,), jnp.int32),
            pltpu.VMEM((_C,), jnp.int32),
            pltpu.VMEM((_D, _C), jnp.int32),
            pltpu.VMEM((_D, _C), jnp.int32),
            pltpu.VMEM((_D, _C), jnp.float32),
            pltpu.VMEM((_D, _C), jnp.float32),
            pltpu.VMEM((_C,), jnp.float32),
            pltpu.VMEM((_C,), jnp.float32),
            pltpu.VMEM((_BPW,), jnp.float32),
            pltpu.SemaphoreType.DMA,
        ],
        compiler_params=pltpu.CompilerParams(needs_layout_passes=False),
    )
    out = run(inputs_t, ut_f, it_f, user_bias_table.T, item_bias_table.T)
    return out.reshape(_B, 1)


# speedup vs baseline: 2.7443x; 2.7443x over previous
import jax
import jax.numpy as jnp
from jax import lax
from jax.experimental import pallas as pl
from jax.experimental.pallas import tpu as pltpu
from jax.experimental.pallas import tpu_sc as plsc

_NC, _NS, _L = 2, 16, 16
_NW = _NC * _NS
_B = 16384
_BPW = _B // _NW
_D = 32
_G = 128 // _D
_C = 128
_NCHUNK = _BPW // _C
_GLOBAL_MEAN = 3.5


def _body(in_hbm, ut_hbm, it_hbm, ub_hbm, ib_hbm, out_hbm,
          uidx, iidx, ugrp, igrp, urows, irows, ubias, ibias, acc, sem):
    wid = lax.axis_index("s") * _NC + lax.axis_index("c")
    base = wid * _BPW

    pltpu.sync_copy(in_hbm.at[0].at[pl.ds(base, _BPW)], uidx)
    pltpu.sync_copy(in_hbm.at[1].at[pl.ds(base, _BPW)], iidx)

    @pl.loop(0, _BPW // _L)
    def _g(b):
        sl = pl.ds(b * _L, _L)
        ugrp[sl] = jnp.right_shift(uidx[sl], 2)
        igrp[sl] = jnp.right_shift(iidx[sl], 2)

    for q in range(_NCHUNK):
        sl = pl.ds(q * _C, _C)
        copies = [
            pltpu.async_copy(ut_hbm.at[ugrp.at[sl]], urows, sem),
            pltpu.async_copy(it_hbm.at[igrp.at[sl]], irows, sem),
            pltpu.async_copy(ub_hbm.at[0].at[uidx.at[sl]], ubias, sem),
            pltpu.async_copy(ib_hbm.at[0].at[iidx.at[sl]], ibias, sem),
        ]
        for c in copies:
            c.wait()

        @pl.loop(0, _C // _L)
        def _blk(b):
            csl = pl.ds(b * _L, _L)
            gsl = pl.ds(q * _C + b * _L, _L)
            rows = lax.iota(jnp.int32, _L) + b * _L
            uoff = jnp.left_shift(jnp.bitwise_and(uidx[gsl], _G - 1), 5)
            ioff = jnp.left_shift(jnp.bitwise_and(iidx[gsl], _G - 1), 5)
            accv = ubias[csl] + ibias[csl] + _GLOBAL_MEAN
            for d in range(_D):
                u = plsc.load_gather(urows, [rows, uoff + d])
                v = plsc.load_gather(irows, [rows, ioff + d])
                accv = accv + u * v
            acc[gsl] = accv

    pltpu.sync_copy(acc, out_hbm.at[pl.ds(base, _BPW)])


def kernel(inputs, user_table, item_table, user_bias_table, item_bias_table):
    inputs_t = inputs.T.astype(jnp.int32)
    ut_g = user_table.reshape(-1, 128)
    it_g = item_table.reshape(-1, 128)
    mesh = plsc.VectorSubcoreMesh(core_axis_name="c", subcore_axis_name="s")
    run = pl.kernel(
        _body,
        out_type=jax.ShapeDtypeStruct((_B,), jnp.float32),
        mesh=mesh,
        scratch_types=[
            pltpu.VMEM((_BPW,), jnp.int32),
            pltpu.VMEM((_BPW,), jnp.int32),
            pltpu.VMEM((_BPW,), jnp.int32),
            pltpu.VMEM((_BPW,), jnp.int32),
            pltpu.VMEM((_C, 128), jnp.float32),
            pltpu.VMEM((_C, 128), jnp.float32),
            pltpu.VMEM((_C,), jnp.float32),
            pltpu.VMEM((_C,), jnp.float32),
            pltpu.VMEM((_BPW,), jnp.float32),
            pltpu.SemaphoreType.DMA,
        ],
        compiler_params=pltpu.CompilerParams(needs_layout_passes=False),
    )
    out = run(inputs_t, ut_g, it_g, user_bias_table.T, item_bias_table.T)
    return out.reshape(_B, 1)
